# baseline (speedup 1.0000x reference)
import jax
import jax.numpy as jnp
from jax.experimental import pallas as pl

_B, _NQ, _NKV, _DIM, _H = 256, 8, 128, 512, 8
_HD = _DIM // _H
_BLK = _NKV // _NQ
_BB = 8


def _fused_body(q_ref, k_ref, v_ref, wkT_ref, wvT_ref, wpT_ref, bp_ref, out_ref):
    R = _BB * _NKV
    QR = _BB * _NQ
    scale = _HD ** -0.5

    kb = k_ref[...].reshape(R, _DIM).astype(jnp.bfloat16)
    qb = q_ref[...].reshape(QR, _DIM).astype(jnp.bfloat16)
    kp = jnp.dot(kb, wkT_ref[...], preferred_element_type=jnp.float32)
    qp = jnp.dot(qb, wkT_ref[...], preferred_element_type=jnp.float32) * scale

    qe = jnp.broadcast_to(qp.reshape(QR, 1, _DIM), (QR, _BLK, _DIM)).reshape(R, _DIM)
    prod = kp * qe
    logits = prod.reshape(R, _H, _HD).sum(axis=-1)

    lg = logits.reshape(QR, _BLK, _H)
    m = jnp.max(lg, axis=1, keepdims=True)
    e = jnp.exp(lg - m)
    s = jnp.sum(e, axis=1, keepdims=True)
    attn = (e / s).reshape(R, _H)

    vb = v_ref[...].reshape(R, _DIM).astype(jnp.bfloat16)
    vp = jnp.dot(vb, wvT_ref[...], preferred_element_type=jnp.float32)
    ae = jnp.broadcast_to(attn.reshape(R, _H, 1), (R, _H, _HD)).reshape(R, _DIM)
    w = vp * ae
    x = w.reshape(QR, _BLK, _DIM).sum(axis=1)

    out = jnp.dot(x.astype(jnp.bfloat16), wpT_ref[...],
                  preferred_element_type=jnp.float32) + bp_ref[...]
    out_ref[...] = out.reshape(_BB, _NQ, _DIM)


def kernel(q, k, v, Wk, Wv, Wp, bp, attn_mask):
    del attn_mask
    wkT = Wk.T.astype(jnp.bfloat16)
    wvT = Wv.T.astype(jnp.bfloat16)
    wpT = Wp.T.astype(jnp.bfloat16)
    bp2 = bp.reshape(1, _DIM)
    return pl.pallas_call(
        _fused_body,
        grid=(_B // _BB,),
        in_specs=[
            pl.BlockSpec((_BB, _NQ, _DIM), lambda i: (i, 0, 0)),
            pl.BlockSpec((_BB, _NKV, _DIM), lambda i: (i, 0, 0)),
            pl.BlockSpec((_BB, _NKV, _DIM), lambda i: (i, 0, 0)),
            pl.BlockSpec((_DIM, _DIM), lambda i: (0, 0)),
            pl.BlockSpec((_DIM, _DIM), lambda i: (0, 0)),
            pl.BlockSpec((_DIM, _DIM), lambda i: (0, 0)),
            pl.BlockSpec((1, _DIM), lambda i: (0, 0)),
        ],
        out_specs=pl.BlockSpec((_BB, _NQ, _DIM), lambda i: (i, 0, 0)),
        out_shape=jax.ShapeDtypeStruct((_B, _NQ, _DIM), jnp.float32),
    )(q, k, v, wkT, wvT, wpT, bp2)


# speedup vs baseline: 1.7832x; 1.7832x over previous
import jax
import jax.numpy as jnp
from jax.experimental import pallas as pl

_B, _NQ, _NKV, _DIM, _H = 256, 8, 128, 512, 8
_HD = _DIM // _H
_BLK = _NKV // _NQ
_BB = 8


def _fused_body(q_ref, k_ref, v_ref, wkT_ref, wvT_ref, wpT_ref, bp_ref,
                e_ref, s_ref, st_ref, g_ref, out_ref):
    R = _BB * _NKV
    QR = _BB * _NQ
    scale = _HD ** -0.5

    kb = k_ref[...].reshape(R, _DIM).astype(jnp.bfloat16)
    qb = q_ref[...].reshape(QR, _DIM).astype(jnp.bfloat16)
    kp = jnp.dot(kb, wkT_ref[...], preferred_element_type=jnp.float32)
    qp = jnp.dot(qb, wkT_ref[...], preferred_element_type=jnp.float32) * scale

    qe = jnp.dot(e_ref[...], qp.astype(jnp.bfloat16),
                 preferred_element_type=jnp.float32)
    prod = (kp * qe).astype(jnp.bfloat16)
    logits = jnp.dot(prod, s_ref[...],
                     preferred_element_type=jnp.float32)

    lg = logits.reshape(QR, _BLK, _H)
    m = jnp.max(lg, axis=1, keepdims=True)
    e = jnp.exp(lg - m)
    s = jnp.sum(e, axis=1, keepdims=True)
    attn = (e / s).reshape(R, _H)

    vb = v_ref[...].reshape(R, _DIM).astype(jnp.bfloat16)
    vp = jnp.dot(vb, wvT_ref[...], preferred_element_type=jnp.float32)
    ae = jnp.dot(attn.astype(jnp.bfloat16), st_ref[...],
                 preferred_element_type=jnp.float32)
    w = (vp * ae).astype(jnp.bfloat16)
    x = jnp.dot(g_ref[...], w, preferred_element_type=jnp.float32)

    out = jnp.dot(x.astype(jnp.bfloat16), wpT_ref[...],
                  preferred_element_type=jnp.float32) + bp_ref[...]
    out_ref[...] = out.reshape(_BB, _NQ, _DIM)


def kernel(q, k, v, Wk, Wv, Wp, bp, attn_mask):
    del attn_mask
    wkT = Wk.T.astype(jnp.bfloat16)
    wvT = Wv.T.astype(jnp.bfloat16)
    wpT = Wp.T.astype(jnp.bfloat16)
    bp2 = bp.reshape(1, _DIM)
    R = _BB * _NKV
    QR = _BB * _NQ
    rows = jnp.arange(R)
    E = (rows[:, None] // _BLK == jnp.arange(QR)[None, :]).astype(jnp.bfloat16)
    S = (jnp.arange(_DIM)[:, None] // _HD == jnp.arange(_H)[None, :]).astype(jnp.bfloat16)
    ST = S.T
    G = E.T
    return pl.pallas_call(
        _fused_body,
        grid=(_B // _BB,),
        in_specs=[
            pl.BlockSpec((_BB, _NQ, _DIM), lambda i: (i, 0, 0)),
            pl.BlockSpec((_BB, _NKV, _DIM), lambda i: (i, 0, 0)),
            pl.BlockSpec((_BB, _NKV, _DIM), lambda i: (i, 0, 0)),
            pl.BlockSpec((_DIM, _DIM), lambda i: (0, 0)),
            pl.BlockSpec((_DIM, _DIM), lambda i: (0, 0)),
            pl.BlockSpec((_DIM, _DIM), lambda i: (0, 0)),
            pl.BlockSpec((1, _DIM), lambda i: (0, 0)),
            pl.BlockSpec((R, QR), lambda i: (0, 0)),
            pl.BlockSpec((_DIM, _H), lambda i: (0, 0)),
            pl.BlockSpec((_H, _DIM), lambda i: (0, 0)),
            pl.BlockSpec((QR, R), lambda i: (0, 0)),
        ],
        out_specs=pl.BlockSpec((_BB, _NQ, _DIM), lambda i: (i, 0, 0)),
        out_shape=jax.ShapeDtypeStruct((_B, _NQ, _DIM), jnp.float32),
    )(q, k, v, wkT, wvT, wpT, bp2, E, S, ST, G)


# speedup vs baseline: 1.9574x; 1.0977x over previous
import jax
import jax.numpy as jnp
from jax.experimental import pallas as pl

_B, _NQ, _NKV, _DIM, _H = 256, 8, 128, 512, 8
_HD = _DIM // _H
_BLK = _NKV // _NQ
_BB = 16


def _fused_body(q_ref, k_ref, v_ref, wkT_ref, wvT_ref, wpT_ref, bp_ref,
                e_ref, s_ref, st_ref, g_ref, out_ref):
    R = _BB * _NKV
    QR = _BB * _NQ
    scale = _HD ** -0.5

    kb = k_ref[...].reshape(R, _DIM).astype(jnp.bfloat16)
    qb = q_ref[...].reshape(QR, _DIM).astype(jnp.bfloat16)
    kp = jnp.dot(kb, wkT_ref[...],
                 preferred_element_type=jnp.float32).astype(jnp.bfloat16)
    qp = (jnp.dot(qb, wkT_ref[...], preferred_element_type=jnp.float32)
          * scale).astype(jnp.bfloat16)

    qe = jnp.dot(e_ref[...], qp,
                 preferred_element_type=jnp.float32).astype(jnp.bfloat16)
    prod = kp * qe
    logits = jnp.dot(prod, s_ref[...],
                     preferred_element_type=jnp.float32)

    lg = logits.reshape(QR, _BLK, _H)
    m = jnp.max(lg, axis=1, keepdims=True)
    e = jnp.exp(lg - m)
    s = jnp.sum(e, axis=1, keepdims=True)
    attn = (e / s).reshape(R, _H)

    vb = v_ref[...].reshape(R, _DIM).astype(jnp.bfloat16)
    vp = jnp.dot(vb, wvT_ref[...],
                 preferred_element_type=jnp.float32).astype(jnp.bfloat16)
    ae = jnp.dot(attn.astype(jnp.bfloat16), st_ref[...],
                 preferred_element_type=jnp.float32).astype(jnp.bfloat16)
    w = vp * ae
    x = jnp.dot(g_ref[...], w, preferred_element_type=jnp.float32)

    out = jnp.dot(x.astype(jnp.bfloat16), wpT_ref[...],
                  preferred_element_type=jnp.float32) + bp_ref[...]
    out_ref[...] = out.reshape(_BB, _NQ, _DIM)


def kernel(q, k, v, Wk, Wv, Wp, bp, attn_mask):
    del attn_mask
    wkT = Wk.T.astype(jnp.bfloat16)
    wvT = Wv.T.astype(jnp.bfloat16)
    wpT = Wp.T.astype(jnp.bfloat16)
    bp2 = bp.reshape(1, _DIM)
    R = _BB * _NKV
    QR = _BB * _NQ
    rows = jnp.arange(R)
    E = (rows[:, None] // _BLK == jnp.arange(QR)[None, :]).astype(jnp.bfloat16)
    S = (jnp.arange(_DIM)[:, None] // _HD == jnp.arange(_H)[None, :]).astype(jnp.bfloat16)
    ST = S.T
    G = E.T
    return pl.pallas_call(
        _fused_body,
        grid=(_B // _BB,),
        in_specs=[
            pl.BlockSpec((_BB, _NQ, _DIM), lambda i: (i, 0, 0)),
            pl.BlockSpec((_BB, _NKV, _DIM), lambda i: (i, 0, 0)),
            pl.BlockSpec((_BB, _NKV, _DIM), lambda i: (i, 0, 0)),
            pl.BlockSpec((_DIM, _DIM), lambda i: (0, 0)),
            pl.BlockSpec((_DIM, _DIM), lambda i: (0, 0)),
            pl.BlockSpec((_DIM, _DIM), lambda i: (0, 0)),
            pl.BlockSpec((1, _DIM), lambda i: (0, 0)),
            pl.BlockSpec((R, QR), lambda i: (0, 0)),
            pl.BlockSpec((_DIM, _H), lambda i: (0, 0)),
            pl.BlockSpec((_H, _DIM), lambda i: (0, 0)),
            pl.BlockSpec((QR, R), lambda i: (0, 0)),
        ],
        out_specs=pl.BlockSpec((_BB, _NQ, _DIM), lambda i: (i, 0, 0)),
        out_shape=jax.ShapeDtypeStruct((_B, _NQ, _DIM), jnp.float32),
    )(q, k, v, wkT, wvT, wpT, bp2, E, S, ST, G)
